# baseline (speedup 1.0000x reference)
import functools

import jax
import jax.numpy as jnp
from jax import lax
from jax.experimental import pallas as pl
from jax.experimental.pallas import tpu as pltpu
from jax.experimental.pallas import tpu_sc as plsc

_N = 1000000
_BATCH = 16384
_D = 16
_L = 16
_NC = 2
_NS = 16
_NW = _NC * _NS
_BPW = _BATCH // _NW
_CHUNK = 128
_NCHUNK = _BPW // _CHUNK
_EPS = 1e-5
_LN2 = 0.6931471805599453

_CW = 1024
_NFULL = _N // _CW
_TAIL = _N - _NFULL * _CW


def _shr(i, n):
    return lax.shift_right_logical(i, jnp.full(i.shape, n, jnp.int32))


def _sqrt16(x):
    i = lax.bitcast_convert_type(x, jnp.int32)
    y = lax.bitcast_convert_type(jnp.int32(0x5F3759DF) - _shr(i, 1),
                                 jnp.float32)
    for _ in range(3):
        y = y * (1.5 - 0.5 * x * y * y)
    return x * y


def _ln16(z):
    i = lax.bitcast_convert_type(z, jnp.int32)
    e = _shr(i, 23) - 127
    m = lax.bitcast_convert_type((i & 0x007FFFFF) | 0x3F800000, jnp.float32)
    big = m > 1.4142135623730951
    m = jnp.where(big, m * 0.5, m)
    ef = (e + jnp.where(big, 1, 0)).astype(jnp.float32)
    s = (m - 1.0) / (m + 1.0)
    s2 = s * s
    p = 2.0 + s2 * (0.66666666666 + s2 * (0.4 + s2 * 0.28571428571))
    return ef * _LN2 + s * p


def _extract(b, ob, nitems):
    step = lax.iota(jnp.int32, _L) * _D
    idxs = [step + c for c in range(_D)]
    ngroups = nitems // _L

    def ld(g, c):
        return b[c, pl.ds(_L * g, _L)]

    cur = [ld(0, c) for c in range(_D)]
    for g in range(ngroups):
        obs = ob.at[pl.ds(g * _L * _D, _L * _D)]
        nxt = []
        for c in range(_D):
            if g + 1 < ngroups:
                nxt.append(ld(g + 1, c))
            plsc.store_scatter(obs, [idxs[c]], cur[c])
        cur = nxt


def _relayout_body(thetaT_hbm, tail_hbm, out_hbm, ba, bb,
                   oba, obb, bt, sra, srb, swa, swb):
    cid = lax.axis_index("c")
    sid = lax.axis_index("s")
    wid = sid * _NC + cid
    start = 32 * wid - 2 * jnp.maximum(wid - 8, 0)
    npairs = jnp.where(wid < 8, 16, 15)

    def _reads(chunk, d, sem):
        off = pl.multiple_of(chunk * _CW, _CW)
        pltpu.async_copy(thetaT_hbm.at[:, pl.ds(off, _CW)], d, sem)

    def _drain_reads(d, sem):
        pltpu.make_async_copy(thetaT_hbm.at[:, pl.ds(0, _CW)],
                              d, sem).wait()

    def _write(chunk, ob, sem):
        off = pl.multiple_of(chunk * _CW * _D, _CW * _D)
        pltpu.async_copy(ob, out_hbm.at[pl.ds(off, _CW * _D)], sem)

    def _drain_write(ob, sem):
        pltpu.make_async_copy(ob, out_hbm.at[pl.ds(0, _CW * _D)], sem).wait()

    _reads(start, ba, sra)

    def body(k, carry):
        ca = start + 2 * k
        cb = ca + 1
        _reads(cb, bb, srb)
        _drain_reads(ba, sra)

        @pl.when(k > 0)
        def _():
            _drain_write(oba, swa)

        _extract(ba, oba, _CW)
        _write(ca, oba, swa)

        @pl.when(k + 1 < npairs)
        def _():
            _reads(ca + 2, ba, sra)

        _drain_reads(bb, srb)

        @pl.when(k > 0)
        def _():
            _drain_write(obb, swb)

        _extract(bb, obb, _CW)
        _write(cb, obb, swb)
        return carry

    lax.fori_loop(0, npairs, body, 0)
    _drain_write(oba, swa)
    _drain_write(obb, swb)

    @pl.when(wid == _NW - 1)
    def _():
        t0 = _NFULL * _CW
        pltpu.sync_copy(thetaT_hbm.at[:, pl.ds(t0, 512)], bt)
        _extract(bt, oba, 512)
        pltpu.sync_copy(oba.at[pl.ds(0, 512 * _D)],
                        out_hbm.at[pl.ds(t0 * _D, 512 * _D)])
        t1 = t0 + 512
        pltpu.sync_copy(tail_hbm, obb.at[pl.ds(0, 64 * _D)])
        pltpu.sync_copy(obb.at[pl.ds(0, 64 * _D)],
                        out_hbm.at[pl.ds(t1 * _D, 64 * _D)])


@functools.cache
def _relayout():
    mesh = plsc.VectorSubcoreMesh(core_axis_name="c", subcore_axis_name="s",
                                  num_cores=_NC, num_subcores=_NS)
    return pl.kernel(
        _relayout_body,
        out_type=jax.ShapeDtypeStruct((_N * _D,), jnp.float32),
        mesh=mesh,
        scratch_types=[
            pltpu.VMEM((_D, _CW), jnp.float32),
            pltpu.VMEM((_D, _CW), jnp.float32),
            pltpu.VMEM((_CW * _D,), jnp.float32),
            pltpu.VMEM((_CW * _D,), jnp.float32),
            pltpu.VMEM((_D, 512), jnp.float32),
            pltpu.SemaphoreType.DMA,
            pltpu.SemaphoreType.DMA,
            pltpu.SemaphoreType.DMA,
            pltpu.SemaphoreType.DMA,
        ],
        compiler_params=pltpu.CompilerParams(needs_layout_passes=False,
                                             use_tc_tiling_on_sc=True),
    )


def _group(rows_u, rows_v, g, r16, t16):
    ri = lax.iota(jnp.int32, _L) + g * _L
    su = jnp.zeros((_L,), jnp.float32)
    sv = jnp.zeros((_L,), jnp.float32)
    sd = jnp.zeros((_L,), jnp.float32)
    for j in range(_D):
        cj = jnp.full((_L,), j, jnp.int32)
        cu = plsc.load_gather(rows_u, [ri, cj])
        cv = plsc.load_gather(rows_v, [ri, cj])
        su = su + cu * cu
        sv = sv + cv * cv
        d = cu - cv
        sd = sd + d * d
    omu = 1.0 - jnp.clip(su, 0.0, 1.0 - _EPS)
    omv = 1.0 - jnp.clip(sv, 0.0, 1.0 - _EPS)
    q = 2.0 * _sqrt16(sd + _EPS) / (omu * omv)
    duv = _ln16(1.0 + q + _sqrt16(q * (q + 2.0)))
    return 1.0 / (jnp.exp((duv - r16) / t16) + 1.0)


def _gather_body(u_hbm, v_hbm, theta_hbm, r_hbm, t_hbm, out_hbm,
                 idx_u, idx_v, rows_u, rows_v, out_v, r_v, t_v, sem):
    cid = lax.axis_index("c")
    sid = lax.axis_index("s")
    wid = sid * _NC + cid
    base = wid * _BPW
    pltpu.sync_copy(r_hbm, r_v)
    pltpu.sync_copy(t_hbm, t_v)
    for c in range(_NCHUNK):
        pltpu.sync_copy(u_hbm.at[pl.ds(base + c * _CHUNK, _CHUNK)],
                        idx_u.at[c])
        pltpu.sync_copy(v_hbm.at[pl.ds(base + c * _CHUNK, _CHUNK)],
                        idx_v.at[c])
    cps = []
    for c in range(_NCHUNK):
        dst_u = rows_u.at[pl.ds(c * _CHUNK, _CHUNK)]
        dst_v = rows_v.at[pl.ds(c * _CHUNK, _CHUNK)]
        cps.append(pltpu.async_copy(theta_hbm.at[idx_u.at[c]], dst_u, sem))
        cps.append(pltpu.async_copy(theta_hbm.at[idx_v.at[c]], dst_v, sem))

    gpc = _CHUNK // _L

    def gbody(g, carry):
        res = _group(rows_u, rows_v, g, r_v[...], t_v[...])
        out_v[pl.ds(g * _L, _L)] = res
        return carry

    for c in range(_NCHUNK):
        cps[2 * c].wait()
        cps[2 * c + 1].wait()
        lax.fori_loop(c * gpc, (c + 1) * gpc, gbody, 0)
    pltpu.sync_copy(out_v, out_hbm.at[pl.ds(base, _BPW)])


@functools.cache
def _poincare_sc():
    mesh = plsc.VectorSubcoreMesh(core_axis_name="c", subcore_axis_name="s",
                                  num_cores=_NC, num_subcores=_NS)
    return pl.kernel(
        _gather_body,
        out_type=jax.ShapeDtypeStruct((_BATCH,), jnp.float32),
        mesh=mesh,
        scratch_types=[
            pltpu.VMEM((_NCHUNK, _CHUNK), jnp.int32),
            pltpu.VMEM((_NCHUNK, _CHUNK), jnp.int32),
            pltpu.VMEM((_BPW, _D), jnp.float32),
            pltpu.VMEM((_BPW, _D), jnp.float32),
            pltpu.VMEM((_BPW,), jnp.float32),
            pltpu.VMEM((_L,), jnp.float32),
            pltpu.VMEM((_L,), jnp.float32),
            pltpu.SemaphoreType.DMA,
        ],
        compiler_params=pltpu.CompilerParams(needs_layout_passes=False,
                                             use_tc_tiling_on_sc=False),
    )


def kernel(u, v, theta, r, t):
    r16 = jnp.broadcast_to(jnp.reshape(r, (1,)).astype(jnp.float32), (_L,))
    t16 = jnp.broadcast_to(jnp.reshape(t, (1,)).astype(jnp.float32), (_L,))
    tail = jnp.reshape(theta[_NFULL * _CW + 512:, :], (64 * _D,))
    theta_flat = _relayout()(theta.T, tail)
    theta_rm = jnp.reshape(theta_flat, (_N, _D))
    return _poincare_sc()(u.astype(jnp.int32), v.astype(jnp.int32),
                          theta_rm, r16, t16)


# speedup vs baseline: 1.0302x; 1.0302x over previous
import functools

import jax
import jax.numpy as jnp
from jax import lax
from jax.experimental import pallas as pl
from jax.experimental.pallas import tpu as pltpu
from jax.experimental.pallas import tpu_sc as plsc

_N = 1000000
_BATCH = 16384
_D = 16
_L = 16
_NC = 2
_NS = 16
_NW = _NC * _NS
_BPW = _BATCH // _NW
_CHUNK = 128
_NCHUNK = _BPW // _CHUNK
_EPS = 1e-5
_LN2 = 0.6931471805599453

_CW = 1024
_NFULL = _N // _CW
_TAIL = _N - _NFULL * _CW


def _shr(i, n):
    return lax.shift_right_logical(i, jnp.full(i.shape, n, jnp.int32))


def _sqrt16(x):
    i = lax.bitcast_convert_type(x, jnp.int32)
    y = lax.bitcast_convert_type(jnp.int32(0x5F3759DF) - _shr(i, 1),
                                 jnp.float32)
    for _ in range(3):
        y = y * (1.5 - 0.5 * x * y * y)
    return x * y


def _ln16(z):
    i = lax.bitcast_convert_type(z, jnp.int32)
    e = _shr(i, 23) - 127
    m = lax.bitcast_convert_type((i & 0x007FFFFF) | 0x3F800000, jnp.float32)
    big = m > 1.4142135623730951
    m = jnp.where(big, m * 0.5, m)
    ef = (e + jnp.where(big, 1, 0)).astype(jnp.float32)
    s = (m - 1.0) / (m + 1.0)
    s2 = s * s
    p = 2.0 + s2 * (0.66666666666 + s2 * (0.4 + s2 * 0.28571428571))
    return ef * _LN2 + s * p


def _extract(b, ob, nitems):
    step = lax.iota(jnp.int32, _L) * _D
    idxs = [step + c for c in range(_D)]
    ngroups = nitems // _L

    def ld(g, c):
        return b[c, pl.ds(_L * g, _L)]

    cur = [ld(0, c) for c in range(_D)]
    for g in range(ngroups):
        obs = ob.at[pl.ds(g * _L * _D, _L * _D)]
        nxt = []
        for c in range(_D):
            if g + 1 < ngroups:
                nxt.append(ld(g + 1, c))
            plsc.store_scatter(obs, [idxs[c]], cur[c])
        cur = nxt


def _relayout_body(thetaT_hbm, tail_hbm, out_hbm, ba, bb,
                   oba, obb, bt, sra, srb, swa, swb):
    cid = lax.axis_index("c")
    sid = lax.axis_index("s")
    wid = sid * _NC + cid
    start = 32 * wid - 2 * jnp.maximum(wid - 8, 0)
    npairs = jnp.where(wid < 8, 16, 15)

    def _reads(chunk, d, sem):
        off = pl.multiple_of(chunk * _CW, _CW)
        pltpu.async_copy(thetaT_hbm.at[:, pl.ds(off, _CW)], d, sem)

    def _drain_reads(d, sem):
        pltpu.make_async_copy(thetaT_hbm.at[:, pl.ds(0, _CW)],
                              d, sem).wait()

    def _write(chunk, ob, sem):
        off = pl.multiple_of(chunk * _CW * _D, _CW * _D)
        pltpu.async_copy(ob, out_hbm.at[pl.ds(off, _CW * _D)], sem)

    def _drain_write(ob, sem):
        pltpu.make_async_copy(ob, out_hbm.at[pl.ds(0, _CW * _D)], sem).wait()

    _reads(start, ba, sra)

    def body(k, carry):
        ca = start + 2 * k
        cb = ca + 1
        _reads(cb, bb, srb)
        _drain_reads(ba, sra)

        @pl.when(k > 0)
        def _():
            _drain_write(oba, swa)

        _extract(ba, oba, _CW)
        _write(ca, oba, swa)

        @pl.when(k + 1 < npairs)
        def _():
            _reads(ca + 2, ba, sra)

        _drain_reads(bb, srb)

        @pl.when(k > 0)
        def _():
            _drain_write(obb, swb)

        _extract(bb, obb, _CW)
        _write(cb, obb, swb)
        return carry

    lax.fori_loop(0, npairs, body, 0)
    _drain_write(oba, swa)
    _drain_write(obb, swb)

    @pl.when(wid == _NW - 1)
    def _():
        t0 = _NFULL * _CW
        pltpu.sync_copy(thetaT_hbm.at[:, pl.ds(t0, 512)], bt)
        _extract(bt, oba, 512)
        pltpu.sync_copy(oba.at[pl.ds(0, 512 * _D)],
                        out_hbm.at[pl.ds(t0 * _D, 512 * _D)])
        t1 = t0 + 512
        pltpu.sync_copy(tail_hbm, obb.at[pl.ds(0, 64 * _D)])
        pltpu.sync_copy(obb.at[pl.ds(0, 64 * _D)],
                        out_hbm.at[pl.ds(t1 * _D, 64 * _D)])


@functools.cache
def _relayout():
    mesh = plsc.VectorSubcoreMesh(core_axis_name="c", subcore_axis_name="s",
                                  num_cores=_NC, num_subcores=_NS)
    return pl.kernel(
        _relayout_body,
        out_type=jax.ShapeDtypeStruct((_N * _D,), jnp.float32),
        mesh=mesh,
        scratch_types=[
            pltpu.VMEM((_D, _CW), jnp.float32),
            pltpu.VMEM((_D, _CW), jnp.float32),
            pltpu.VMEM((_CW * _D,), jnp.float32),
            pltpu.VMEM((_CW * _D,), jnp.float32),
            pltpu.VMEM((_D, 512), jnp.float32),
            pltpu.SemaphoreType.DMA,
            pltpu.SemaphoreType.DMA,
            pltpu.SemaphoreType.DMA,
            pltpu.SemaphoreType.DMA,
        ],
        compiler_params=pltpu.CompilerParams(needs_layout_passes=False,
                                             use_tc_tiling_on_sc=True),
    )


def _group(rows_u, rows_v, g, r16, t16):
    ri = lax.iota(jnp.int32, _L) + g * _L
    su = jnp.zeros((_L,), jnp.float32)
    sv = jnp.zeros((_L,), jnp.float32)
    sd = jnp.zeros((_L,), jnp.float32)
    for j in range(_D):
        cj = jnp.full((_L,), j, jnp.int32)
        cu = plsc.load_gather(rows_u, [ri, cj])
        cv = plsc.load_gather(rows_v, [ri, cj])
        su = su + cu * cu
        sv = sv + cv * cv
        d = cu - cv
        sd = sd + d * d
    omu = 1.0 - jnp.clip(su, 0.0, 1.0 - _EPS)
    omv = 1.0 - jnp.clip(sv, 0.0, 1.0 - _EPS)
    q = 2.0 * _sqrt16(sd + _EPS) / (omu * omv)
    duv = _ln16(1.0 + q + _sqrt16(q * (q + 2.0)))
    return 1.0 / (jnp.exp((duv - r16) / t16) + 1.0)


def _gather_body(u_hbm, v_hbm, theta_hbm, r_hbm, t_hbm, out_hbm,
                 idx_u, idx_v, rows_u, rows_v, out_v, r_v, t_v, sem):
    cid = lax.axis_index("c")
    sid = lax.axis_index("s")
    wid = sid * _NC + cid
    base = wid * _BPW
    pltpu.sync_copy(r_hbm, r_v)
    pltpu.sync_copy(t_hbm, t_v)
    for c in range(_NCHUNK):
        pltpu.sync_copy(u_hbm.at[pl.ds(base + c * _CHUNK, _CHUNK)],
                        idx_u.at[c])
        pltpu.sync_copy(v_hbm.at[pl.ds(base + c * _CHUNK, _CHUNK)],
                        idx_v.at[c])
    cps = []
    for c in range(_NCHUNK):
        dst_u = rows_u.at[pl.ds(c * _CHUNK, _CHUNK)]
        dst_v = rows_v.at[pl.ds(c * _CHUNK, _CHUNK)]
        cps.append(pltpu.async_copy(theta_hbm.at[idx_u.at[c]], dst_u, sem))
        cps.append(pltpu.async_copy(theta_hbm.at[idx_v.at[c]], dst_v, sem))

    for cp in cps:
        cp.wait()

    def gbody(g, carry):
        res = _group(rows_u, rows_v, g, r_v[...], t_v[...])
        out_v[pl.ds(g * _L, _L)] = res
        return carry

    lax.fori_loop(0, _BPW // _L, gbody, 0)
    pltpu.sync_copy(out_v, out_hbm.at[pl.ds(base, _BPW)])


@functools.cache
def _poincare_sc():
    mesh = plsc.VectorSubcoreMesh(core_axis_name="c", subcore_axis_name="s",
                                  num_cores=_NC, num_subcores=_NS)
    return pl.kernel(
        _gather_body,
        out_type=jax.ShapeDtypeStruct((_BATCH,), jnp.float32),
        mesh=mesh,
        scratch_types=[
            pltpu.VMEM((_NCHUNK, _CHUNK), jnp.int32),
            pltpu.VMEM((_NCHUNK, _CHUNK), jnp.int32),
            pltpu.VMEM((_BPW, _D), jnp.float32),
            pltpu.VMEM((_BPW, _D), jnp.float32),
            pltpu.VMEM((_BPW,), jnp.float32),
            pltpu.VMEM((_L,), jnp.float32),
            pltpu.VMEM((_L,), jnp.float32),
            pltpu.SemaphoreType.DMA,
        ],
        compiler_params=pltpu.CompilerParams(needs_layout_passes=False,
                                             use_tc_tiling_on_sc=False),
    )


def kernel(u, v, theta, r, t):
    r16 = jnp.broadcast_to(jnp.reshape(r, (1,)).astype(jnp.float32), (_L,))
    t16 = jnp.broadcast_to(jnp.reshape(t, (1,)).astype(jnp.float32), (_L,))
    tail = jnp.reshape(theta[_NFULL * _CW + 512:, :], (64 * _D,))
    theta_flat = _relayout()(theta.T, tail)
    theta_rm = jnp.reshape(theta_flat, (_N, _D))
    return _poincare_sc()(u.astype(jnp.int32), v.astype(jnp.int32),
                          theta_rm, r16, t16)
